# baseline (speedup 1.0000x reference)
import math

import jax
import jax.numpy as jnp
from jax.experimental import pallas as pl
from jax.experimental.pallas import tpu as pltpu

_H = 512
_W = 512
_C = 3
_CELL = 16
_NB = 9
_HC = _H // _CELL
_WC = _W // _CELL
_GAIN = _NB / math.pi
_BLK = 4


def _one_image(im):
    li = jax.lax.broadcasted_iota(jnp.int32, (1, _W), 1)
    colmask = ((li > 0) & (li < _W - 1)).astype(jnp.float32)
    colsign = (jnp.where(li == 0, 1.0, 0.0)
               - jnp.where(li == _W - 1, 1.0, 0.0))
    ri = jax.lax.broadcasted_iota(jnp.int32, (_H, 1), 0)
    rowmask = ((ri > 0) & (ri < _H - 1)).astype(jnp.float32)
    rowsign = (jnp.where(ri == 0, 1.0, 0.0)
               - jnp.where(ri == _H - 1, 1.0, 0.0))

    gxs, gys, nsq = [], [], []
    for c in range(_C):
        a = im[c].astype(jnp.bfloat16).astype(jnp.float32)
        am = pltpu.roll(a, 1, 1)
        ap = pltpu.roll(a, _W - 1, 1)
        draw = ap - am
        d = draw * colmask
        s = ((am + ap) + 2.0 * a) + draw * colsign
        dm = pltpu.roll(d, 1, 0)
        dp = pltpu.roll(d, _H - 1, 0)
        gx = ((dm + dp) + 2.0 * d) + (dp - dm) * rowsign
        sm = pltpu.roll(s, 1, 0)
        sp = pltpu.roll(s, _H - 1, 0)
        gy = (sp - sm) * rowmask
        gxs.append(gx)
        gys.append(gy)
        nsq.append(gx * gx + gy * gy)

    sel0 = (nsq[0] >= nsq[1]) & (nsq[0] >= nsq[2])
    sel1 = jnp.logical_not(sel0) & (nsq[1] >= nsq[2])
    gx_m = jnp.where(sel0, gxs[0], jnp.where(sel1, gxs[1], gxs[2]))
    gy_m = jnp.where(sel0, gys[0], jnp.where(sel1, gys[1], gys[2]))
    nsq_m = jnp.where(sel0, nsq[0], jnp.where(sel1, nsq[1], nsq[2]))
    norm_m = jnp.sqrt(nsq_m)

    ax = jnp.abs(gx_m)
    ay = jnp.abs(gy_m)
    mx = jnp.maximum(ax, ay)
    mn = jnp.minimum(ax, ay)
    q = mn / jnp.where(mx == 0.0, 1.0, mx)
    z = q * q
    c0, c1, c2, c3, c4 = (_GAIN * 0.9998660, _GAIN * -0.3302995,
                          _GAIN * 0.1801410, _GAIN * -0.0851330,
                          _GAIN * 0.0208351)
    p = q * (c0 + z * (c1 + z * (c2 + z * (c3 + z * c4))))
    acute = jnp.where(ay > ax, 4.5 - p, p)
    x = jnp.where(gx_m * gy_m < 0.0, 9.0 - acute, acute)

    x = jnp.where(x >= 9.0, 0.0, x)
    l = jnp.floor(x)
    f = x - l
    lb = l.astype(jnp.bfloat16)
    v1 = (f * norm_m).astype(jnp.bfloat16)
    v0 = norm_m.astype(jnp.bfloat16) - v1
    zero = jnp.zeros_like(v0)

    hi = jax.lax.broadcasted_iota(jnp.int32, (_HC, _H), 1)
    ii = jax.lax.broadcasted_iota(jnp.int32, (_HC, _H), 0)
    s16t = (hi // _CELL == ii).astype(jnp.bfloat16)
    rs = []
    for b in range(_NB):
        m = (jnp.where(lb == float(b), v0, zero)
             + jnp.where(lb == float((b - 1) % _NB), v1, zero))
        rs.append(jax.lax.dot_general(
            s16t, m, (((1,), (0,)), ((), ())),
            preferred_element_type=jnp.float32))
    r_all = jnp.concatenate(rs, axis=0)

    wi = jax.lax.broadcasted_iota(jnp.int32, (_W, _WC), 0)
    ji = jax.lax.broadcasted_iota(jnp.int32, (_W, _WC), 1)
    sones = (wi // _CELL == ji).astype(jnp.float32)
    hist = jax.lax.dot_general(
        r_all, sones, (((1,), (0,)), ((), ())),
        preferred_element_type=jnp.float32,
        precision=jax.lax.Precision.HIGHEST)
    hog = hist.reshape(_NB, _HC, _WC) * (1.0 / (_CELL * _CELL))

    ssq = jnp.sum(hog * hog, axis=0)
    s4 = (ssq[0:_HC - 1, 0:_WC - 1] + ssq[1:_HC, 0:_WC - 1]
          + ssq[0:_HC - 1, 1:_WC] + ssq[1:_HC, 1:_WC])
    inv_n1 = 1.0 / (jnp.sqrt(s4) + 1e-10)

    ts = []
    n2sq = jnp.zeros((_HC - 1, _WC - 1), jnp.float32)
    for u in (0, 1):
        for v in (0, 1):
            xuv = hog[:, u:u + _HC - 1, v:v + _WC - 1]
            t = jnp.minimum(xuv * inv_n1[None], 0.2)
            ts.append(t)
            n2sq = n2sq + jnp.sum(t * t, axis=0)
    inv_n2 = 1.0 / (jnp.sqrt(n2sq) + 1e-10)

    out = jnp.stack(ts, axis=1) * inv_n2[None, None]
    return out.reshape(_NB * 4, _HC - 1, _WC - 1)


def _hog_body(im_ref, out_ref):
    for i in range(_BLK):
        out_ref[i] = _one_image(im_ref[i])


def kernel(im):
    b = im.shape[0]
    res = pl.pallas_call(
        _hog_body,
        grid=(b // _BLK,),
        in_specs=[pl.BlockSpec((_BLK, _C, _H, _W), lambda i: (i, 0, 0, 0))],
        out_specs=pl.BlockSpec((_BLK, _NB * 4, _HC - 1, _WC - 1),
                               lambda i: (i, 0, 0, 0)),
        out_shape=jax.ShapeDtypeStruct((b, _NB * 4, _HC - 1, _WC - 1),
                                       jnp.float32),
        compiler_params=pltpu.CompilerParams(
            dimension_semantics=("parallel",)),
    )(im)
    res = res.reshape(b, _NB, 2, 2, _HC - 1, _WC - 1)
    res = jnp.transpose(res, (0, 4, 5, 2, 3, 1))
    return res.reshape(b, 1, _HC - 1, _WC - 1, 2, 2, _NB)


# speedup vs baseline: 1.0760x; 1.0760x over previous
import math

import jax
import jax.numpy as jnp
from jax.experimental import pallas as pl
from jax.experimental.pallas import tpu as pltpu

_H = 512
_W = 512
_C = 3
_CELL = 16
_NB = 9
_HC = _H // _CELL
_WC = _W // _CELL
_GAIN = _NB / math.pi
_BLK = 4


def _shift_h(a, d):
    if d == 1:
        return jnp.concatenate([a[1:], a[_H - 2:_H - 1]], axis=0)
    return jnp.concatenate([a[1:2], a[:_H - 1]], axis=0)


def _shift_w(a, d):
    if d == 1:
        return jnp.concatenate([a[:, 1:], a[:, _W - 2:_W - 1]], axis=1)
    return jnp.concatenate([a[:, 1:2], a[:, :_W - 1]], axis=1)


def _one_image(im):
    gxs, gys, nsq = [], [], []
    for c in range(_C):
        a = im[c].astype(jnp.bfloat16).astype(jnp.float32)
        wp1 = _shift_w(a, 1)
        wm1 = _shift_w(a, -1)
        d = wp1 - wm1
        gx = (_shift_h(d, -1) + _shift_h(d, 1)) + 2.0 * d
        s = (wp1 + wm1) + 2.0 * a
        gy = _shift_h(s, 1) - _shift_h(s, -1)
        gxs.append(gx)
        gys.append(gy)
        nsq.append(gx * gx + gy * gy)

    sel0 = (nsq[0] >= nsq[1]) & (nsq[0] >= nsq[2])
    sel1 = jnp.logical_not(sel0) & (nsq[1] >= nsq[2])
    gx_m = jnp.where(sel0, gxs[0], jnp.where(sel1, gxs[1], gxs[2]))
    gy_m = jnp.where(sel0, gys[0], jnp.where(sel1, gys[1], gys[2]))
    nsq_m = jnp.where(sel0, nsq[0], jnp.where(sel1, nsq[1], nsq[2]))
    norm_m = jnp.sqrt(nsq_m)

    ax = jnp.abs(gx_m)
    ay = jnp.abs(gy_m)
    mx = jnp.maximum(ax, ay)
    mn = jnp.minimum(ax, ay)
    q = mn / jnp.where(mx == 0.0, 1.0, mx)
    z = q * q
    c0, c1, c2, c3, c4 = (_GAIN * 0.9998660, _GAIN * -0.3302995,
                          _GAIN * 0.1801410, _GAIN * -0.0851330,
                          _GAIN * 0.0208351)
    p = q * (c0 + z * (c1 + z * (c2 + z * (c3 + z * c4))))
    acute = jnp.where(ay > ax, 4.5 - p, p)
    x = jnp.where(gx_m * gy_m < 0.0, 9.0 - acute, acute)

    x = jnp.where(x >= 9.0, 0.0, x)
    l = jnp.floor(x)
    f = x - l
    lb = l.astype(jnp.bfloat16)
    v1 = (f * norm_m).astype(jnp.bfloat16)
    v0 = norm_m.astype(jnp.bfloat16) - v1
    zero = jnp.zeros_like(v0)

    hi = jax.lax.broadcasted_iota(jnp.int32, (_HC, _H), 1)
    ii = jax.lax.broadcasted_iota(jnp.int32, (_HC, _H), 0)
    s16t = (hi // _CELL == ii).astype(jnp.bfloat16)
    rs = []
    for b in range(_NB):
        m = (jnp.where(lb == float(b), v0, zero)
             + jnp.where(lb == float((b - 1) % _NB), v1, zero))
        rs.append(jax.lax.dot_general(
            s16t, m, (((1,), (0,)), ((), ())),
            preferred_element_type=jnp.float32))
    r_all = jnp.concatenate(rs, axis=0)

    wi = jax.lax.broadcasted_iota(jnp.int32, (_W, _WC), 0)
    ji = jax.lax.broadcasted_iota(jnp.int32, (_W, _WC), 1)
    sones = (wi // _CELL == ji).astype(jnp.float32)
    hist = jax.lax.dot_general(
        r_all, sones, (((1,), (0,)), ((), ())),
        preferred_element_type=jnp.float32,
        precision=jax.lax.Precision.HIGHEST)
    hog = hist.reshape(_NB, _HC, _WC) * (1.0 / (_CELL * _CELL))

    ssq = jnp.sum(hog * hog, axis=0)
    s4 = (ssq[0:_HC - 1, 0:_WC - 1] + ssq[1:_HC, 0:_WC - 1]
          + ssq[0:_HC - 1, 1:_WC] + ssq[1:_HC, 1:_WC])
    inv_n1 = 1.0 / (jnp.sqrt(s4) + 1e-10)

    ts = []
    n2sq = jnp.zeros((_HC - 1, _WC - 1), jnp.float32)
    for u in (0, 1):
        for v in (0, 1):
            xuv = hog[:, u:u + _HC - 1, v:v + _WC - 1]
            t = jnp.minimum(xuv * inv_n1[None], 0.2)
            ts.append(t)
            n2sq = n2sq + jnp.sum(t * t, axis=0)
    inv_n2 = 1.0 / (jnp.sqrt(n2sq) + 1e-10)

    out = jnp.stack(ts, axis=1) * inv_n2[None, None]
    return out.reshape(_NB * 4, _HC - 1, _WC - 1)


def _hog_body(im_ref, out_ref):
    for i in range(_BLK):
        out_ref[i] = _one_image(im_ref[i])


def kernel(im):
    b = im.shape[0]
    res = pl.pallas_call(
        _hog_body,
        grid=(b // _BLK,),
        in_specs=[pl.BlockSpec((_BLK, _C, _H, _W), lambda i: (i, 0, 0, 0))],
        out_specs=pl.BlockSpec((_BLK, _NB * 4, _HC - 1, _WC - 1),
                               lambda i: (i, 0, 0, 0)),
        out_shape=jax.ShapeDtypeStruct((b, _NB * 4, _HC - 1, _WC - 1),
                                       jnp.float32),
        compiler_params=pltpu.CompilerParams(
            dimension_semantics=("parallel",)),
    )(im)
    res = res.reshape(b, _NB, 2, 2, _HC - 1, _WC - 1)
    res = jnp.transpose(res, (0, 4, 5, 2, 3, 1))
    return res.reshape(b, 1, _HC - 1, _WC - 1, 2, 2, _NB)
